# baseline (speedup 1.0000x reference)
import functools

import jax
import jax.numpy as jnp
from jax import lax
from jax.experimental import pallas as pl
from jax.experimental.pallas import tpu as pltpu
from jax.experimental.pallas import tpu_sc as plsc

NC = 2
NS = 16
NW = NC * NS
L = 16
R = 64
RP = 128
B = 16384
BPW = B // NW
CHUNK = 128
NCHUNK = BPW // CHUNK
GPC = CHUNK // L


def _body(idx_hbm, f0_hbm, f1_hbm, f2_hbm, out_hbm,
          idx_v, bufs, out_v, sem):
    wid = lax.axis_index("s") * NC + lax.axis_index("c")
    tables = (f0_hbm, f1_hbm, f2_hbm)

    for t in range(3):
        pltpu.sync_copy(idx_hbm.at[t, pl.ds(wid * NCHUNK, NCHUNK)],
                        idx_v.at[t])

    def fire(c):
        return [pltpu.async_copy(tables[t].at[idx_v.at[t, c]],
                                 bufs[t][c % 2], sem)
                for t in range(3)]

    iota = lax.iota(jnp.int32, L)
    lane15 = jnp.full((L,), L - 1, jnp.int32)
    inflight = fire(0)
    for c in range(NCHUNK):
        nxt = fire(c + 1) if c + 1 < NCHUNK else []
        for cp in inflight:
            cp.wait()
        g0, g1, g2 = (bufs[t][c % 2] for t in range(3))

        @plsc.parallel_loop(0, GPC, unroll=2)
        def group(k):
            acc = jnp.zeros((L,), jnp.float32)
            for e in range(L):
                b = k * L + e
                p = [g0[b, pl.ds(rc * L, L)] * g1[b, pl.ds(rc * L, L)]
                     * g2[b, pl.ds(rc * L, L)] for rc in range(R // L)]
                s = (p[0] + p[1]) + (p[2] + p[3])
                cs = plsc.cumsum(s)
                tot = lax.gather(
                    cs, lane15[:, None],
                    lax.GatherDimensionNumbers(offset_dims=(),
                                               collapsed_slice_dims=(0,),
                                               start_index_map=(0,)),
                    (1,), mode=lax.GatherScatterMode.PROMISE_IN_BOUNDS)
                acc = jnp.where(iota == e, tot, acc)
            out_v[c, pl.ds(k * L, L)] = acc

        inflight = nxt

    pltpu.sync_copy(out_v, out_hbm.at[pl.ds(wid * NCHUNK, NCHUNK)])


@functools.partial(
    pl.kernel,
    out_type=jax.ShapeDtypeStruct((B // RP, RP), jnp.float32),
    mesh=plsc.VectorSubcoreMesh(core_axis_name="c", subcore_axis_name="s",
                                num_cores=NC, num_subcores=NS),
    scratch_types=[
        pltpu.VMEM((3, NCHUNK, CHUNK), jnp.int32),
    ] + [pltpu.VMEM((CHUNK, RP), jnp.float32) for _ in range(6)] + [
        pltpu.VMEM((NCHUNK, CHUNK), jnp.float32),
        pltpu.SemaphoreType.DMA,
    ],
    compiler_params=pltpu.CompilerParams(needs_layout_passes=False,
                                         use_tc_tiling_on_sc=True),
)
def _cp_lookup(idx_hbm, f0_hbm, f1_hbm, f2_hbm, out_hbm,
               idx_v, b0a, b0b, b1a, b1b, b2a, b2b, out_v, sem):
    bufs = ((b0a, b0b), (b1a, b1b), (b2a, b2b))
    _body(idx_hbm, f0_hbm, f1_hbm, f2_hbm, out_hbm,
          idx_v, bufs, out_v, sem)


NROW = 100000
BV = 4096


def _tr_body(in_ref, eye_ref, out_ref):
    out_ref[:, pl.ds(0, R)] = lax.dot_general(
        in_ref[...], eye_ref[...], (((0,), (0,)), ((), ())),
        preferred_element_type=jnp.float32,
        precision=lax.Precision.HIGHEST)


_transpose = pl.pallas_call(
    _tr_body,
    grid=((NROW + BV - 1) // BV,),
    in_specs=[pl.BlockSpec((R, BV), lambda i: (0, i)),
              pl.BlockSpec((R, R), lambda i: (0, 0))],
    out_specs=pl.BlockSpec((BV, RP), lambda i: (i, 0)),
    out_shape=jax.ShapeDtypeStruct((NROW, RP), jnp.float32),
)


def kernel(input, F0, F1, F2):
    idx = jnp.transpose(input.astype(jnp.int32)).reshape(3, B // CHUNK, CHUNK)
    eye = jnp.eye(R, dtype=jnp.float32)
    out2d = _cp_lookup(idx, _transpose(F0.T, eye), _transpose(F1.T, eye),
                       _transpose(F2.T, eye))
    return out2d.reshape(B)


# speedup vs baseline: 1.4763x; 1.4763x over previous
import functools

import jax
import jax.numpy as jnp
from jax import lax
from jax.experimental import pallas as pl
from jax.experimental.pallas import tpu as pltpu
from jax.experimental.pallas import tpu_sc as plsc

NC = 2
NS = 16
NW = NC * NS
L = 16
R = 64
RP = 128
B = 16384
BPW = B // NW
CHUNK = 128
NCHUNK = BPW // CHUNK
GPC = CHUNK // L


def _body(idx_hbm, f0_hbm, f1_hbm, f2_hbm, out_hbm,
          idx_v, bufs, out_v, sem):
    wid = lax.axis_index("s") * NC + lax.axis_index("c")
    tables = (f0_hbm, f1_hbm, f2_hbm)

    for t in range(3):
        pltpu.sync_copy(idx_hbm.at[t, pl.ds(wid * NCHUNK, NCHUNK)],
                        idx_v.at[t])

    def fire(c):
        return [pltpu.async_copy(tables[t].at[idx_v.at[t, c]],
                                 bufs[t][c % 2], sem)
                for t in range(3)]

    iota = lax.iota(jnp.int32, L)
    lane15 = jnp.full((L,), L - 1, jnp.int32)
    inflight = fire(0)
    for c in range(NCHUNK):
        nxt = fire(c + 1) if c + 1 < NCHUNK else []
        for cp in inflight:
            cp.wait()
        g0, g1, g2 = (bufs[t][c % 2] for t in range(3))

        @plsc.parallel_loop(0, GPC, unroll=2)
        def group(k):
            acc = jnp.zeros((L,), jnp.float32)
            for e in range(L):
                b = k * L + e
                p = [g0[b, pl.ds(rc * L, L)] * g1[b, pl.ds(R + rc * L, L)]
                     * g2[b, pl.ds(rc * L, L)] for rc in range(R // L)]
                s = (p[0] + p[1]) + (p[2] + p[3])
                cs = plsc.cumsum(s)
                tot = lax.gather(
                    cs, lane15[:, None],
                    lax.GatherDimensionNumbers(offset_dims=(),
                                               collapsed_slice_dims=(0,),
                                               start_index_map=(0,)),
                    (1,), mode=lax.GatherScatterMode.PROMISE_IN_BOUNDS)
                acc = jnp.where(iota == e, tot, acc)
            out_v[c, pl.ds(k * L, L)] = acc

        inflight = nxt

    pltpu.sync_copy(out_v, out_hbm.at[pl.ds(wid * NCHUNK, NCHUNK)])


@functools.partial(
    pl.kernel,
    out_type=jax.ShapeDtypeStruct((B // RP, RP), jnp.float32),
    mesh=plsc.VectorSubcoreMesh(core_axis_name="c", subcore_axis_name="s",
                                num_cores=NC, num_subcores=NS),
    scratch_types=[
        pltpu.VMEM((3, NCHUNK, CHUNK), jnp.int32),
    ] + [pltpu.VMEM((CHUNK, RP), jnp.float32) for _ in range(6)] + [
        pltpu.VMEM((NCHUNK, CHUNK), jnp.float32),
        pltpu.SemaphoreType.DMA,
    ],
    compiler_params=pltpu.CompilerParams(needs_layout_passes=False,
                                         use_tc_tiling_on_sc=True),
)
def _cp_lookup(idx_hbm, f0_hbm, f1_hbm, f2_hbm, out_hbm,
               idx_v, b0a, b0b, b1a, b1b, b2a, b2b, out_v, sem):
    bufs = ((b0a, b0b), (b1a, b1b), (b2a, b2b))
    _body(idx_hbm, f0_hbm, f1_hbm, f2_hbm, out_hbm,
          idx_v, bufs, out_v, sem)


NROW = 100000
BV = 4096


def _mxu_t(x, eye):
    return lax.dot_general(x, eye, (((0,), (0,)), ((), ())),
                           preferred_element_type=jnp.float32)


def _tr2_body(a_ref, b_ref, eye_ref, out_ref):
    out_ref[:, pl.ds(0, R)] = _mxu_t(a_ref[...], eye_ref[...])
    out_ref[:, pl.ds(R, R)] = _mxu_t(b_ref[...], eye_ref[...])


def _tr1_body(a_ref, eye_ref, out_ref):
    out_ref[:, pl.ds(0, R)] = _mxu_t(a_ref[...], eye_ref[...])


_in_spec = pl.BlockSpec((R, BV), lambda i: (0, i))
_eye_spec = pl.BlockSpec((R, R), lambda i: (0, 0))
_tr_grid = (NROW + BV - 1) // BV
_tr_out = jax.ShapeDtypeStruct((NROW, RP), jnp.float32)
_out_spec = pl.BlockSpec((BV, RP), lambda i: (i, 0))

_transpose2 = pl.pallas_call(
    _tr2_body, grid=(_tr_grid,),
    in_specs=[_in_spec, _in_spec, _eye_spec],
    out_specs=_out_spec, out_shape=_tr_out,
)

_transpose1 = pl.pallas_call(
    _tr1_body, grid=(_tr_grid,),
    in_specs=[_in_spec, _eye_spec],
    out_specs=_out_spec, out_shape=_tr_out,
)


def kernel(input, F0, F1, F2):
    idx = jnp.transpose(input.astype(jnp.int32)).reshape(3, B // CHUNK, CHUNK)
    eye = jnp.eye(R, dtype=jnp.float32)
    g01 = _transpose2(F0.T, F1.T, eye)
    g2 = _transpose1(F2.T, eye)
    out2d = _cp_lookup(idx, g01, g01, g2)
    return out2d.reshape(B)


# speedup vs baseline: 1.6261x; 1.1015x over previous
import functools

import jax
import jax.numpy as jnp
from jax import lax
from jax.experimental import pallas as pl
from jax.experimental.pallas import tpu as pltpu
from jax.experimental.pallas import tpu_sc as plsc

NC = 2
NS = 16
NW = NC * NS
L = 16
R = 64
RP = 128
B = 16384
BPW = B // NW
CHUNK = 128
NCHUNK = BPW // CHUNK
GPC = CHUNK // L


def _body(idx_hbm, f0_hbm, f1_hbm, f2_hbm, out_hbm,
          idx_v, bufs, out_v, sem):
    wid = lax.axis_index("s") * NC + lax.axis_index("c")
    tables = (f0_hbm, f1_hbm, f2_hbm)

    for t in range(3):
        pltpu.sync_copy(idx_hbm.at[t, pl.ds(wid * NCHUNK, NCHUNK)],
                        idx_v.at[t])

    def fire(c):
        return [pltpu.async_copy(tables[t].at[idx_v.at[t, c]],
                                 bufs[t][c % 2], sem)
                for t in range(3)]

    iota = lax.iota(jnp.int32, L)
    lane15 = jnp.full((L,), L - 1, jnp.int32)
    inflight = fire(0)
    for c in range(NCHUNK):
        nxt = fire(c + 1) if c + 1 < NCHUNK else []
        for cp in inflight:
            cp.wait()
        g0, g1, g2 = (bufs[t][c % 2] for t in range(3))

        @plsc.parallel_loop(0, GPC, unroll=2)
        def group(k):
            acc = jnp.zeros((L,), jnp.float32)
            for e in range(L):
                b = k * L + e
                p = [g0[b, pl.ds(rc * L, L)] * g1[b, pl.ds(R + rc * L, L)]
                     * g2[b, pl.ds(rc * L, L)] for rc in range(R // L)]
                s = (p[0] + p[1]) + (p[2] + p[3])
                cs = plsc.cumsum(s)
                tot = lax.gather(
                    cs, lane15[:, None],
                    lax.GatherDimensionNumbers(offset_dims=(),
                                               collapsed_slice_dims=(0,),
                                               start_index_map=(0,)),
                    (1,), mode=lax.GatherScatterMode.PROMISE_IN_BOUNDS)
                acc = jnp.where(iota == e, tot, acc)
            out_v[c, pl.ds(k * L, L)] = acc

        inflight = nxt

    pltpu.sync_copy(out_v, out_hbm.at[pl.ds(wid * NCHUNK, NCHUNK)])


@functools.partial(
    pl.kernel,
    out_type=jax.ShapeDtypeStruct((B // RP, RP), jnp.float32),
    mesh=plsc.VectorSubcoreMesh(core_axis_name="c", subcore_axis_name="s",
                                num_cores=NC, num_subcores=NS),
    scratch_types=[
        pltpu.VMEM((3, NCHUNK, CHUNK), jnp.int32),
    ] + [pltpu.VMEM((CHUNK, RP), jnp.float32) for _ in range(6)] + [
        pltpu.VMEM((NCHUNK, CHUNK), jnp.float32),
        pltpu.SemaphoreType.DMA,
    ],
    compiler_params=pltpu.CompilerParams(needs_layout_passes=False,
                                         use_tc_tiling_on_sc=True),
)
def _cp_lookup(idx_hbm, f0_hbm, f1_hbm, f2_hbm, out_hbm,
               idx_v, b0a, b0b, b1a, b1b, b2a, b2b, out_v, sem):
    bufs = ((b0a, b0b), (b1a, b1b), (b2a, b2b))
    _body(idx_hbm, f0_hbm, f1_hbm, f2_hbm, out_hbm,
          idx_v, bufs, out_v, sem)


NROW = 100000
BV = 8192


def _mxu_t(x, eye):
    return lax.dot_general(x, eye, (((0,), (0,)), ((), ())),
                           preferred_element_type=jnp.float32)


def _tr2_body(a_ref, b_ref, eye_ref, out_ref):
    out_ref[:, pl.ds(0, R)] = _mxu_t(a_ref[...], eye_ref[...])
    out_ref[:, pl.ds(R, R)] = _mxu_t(b_ref[...], eye_ref[...])


def _tr1_body(a_ref, eye_ref, out_ref):
    out_ref[:, pl.ds(0, R)] = _mxu_t(a_ref[...], eye_ref[...])


_in_spec = pl.BlockSpec((R, BV), lambda i: (0, i))
_eye_spec = pl.BlockSpec((R, R), lambda i: (0, 0))
_tr_grid = (NROW + BV - 1) // BV
_tr_out = jax.ShapeDtypeStruct((NROW, RP), jnp.float32)
_out_spec = pl.BlockSpec((BV, RP), lambda i: (i, 0))

_transpose2 = pl.pallas_call(
    _tr2_body, grid=(_tr_grid,),
    in_specs=[_in_spec, _in_spec, _eye_spec],
    out_specs=_out_spec, out_shape=_tr_out,
)

_transpose1 = pl.pallas_call(
    _tr1_body, grid=(_tr_grid,),
    in_specs=[_in_spec, _eye_spec],
    out_specs=_out_spec, out_shape=_tr_out,
)


def kernel(input, F0, F1, F2):
    idx = jnp.transpose(input.astype(jnp.int32)).reshape(3, B // CHUNK, CHUNK)
    eye = jnp.eye(R, dtype=jnp.float32)
    g01 = _transpose2(F0.T, F1.T, eye)
    g2 = _transpose1(F2.T, eye)
    out2d = _cp_lookup(idx, g01, g01, g2)
    return out2d.reshape(B)


# speedup vs baseline: 1.6382x; 1.0074x over previous
import functools

import jax
import jax.numpy as jnp
from jax import lax
from jax.experimental import pallas as pl
from jax.experimental.pallas import tpu as pltpu
from jax.experimental.pallas import tpu_sc as plsc

NC = 2
NS = 16
NW = NC * NS
L = 16
R = 64
RP = 128
B = 16384
BPW = B // NW
CHUNK = 128
NCHUNK = BPW // CHUNK
GPC = CHUNK // L


def _body(idx_hbm, f0_hbm, f1_hbm, f2_hbm, out_hbm,
          idx_v, bufs, out_v, sem):
    wid = lax.axis_index("s") * NC + lax.axis_index("c")
    tables = (f0_hbm, f1_hbm, f2_hbm)

    for t in range(3):
        pltpu.sync_copy(idx_hbm.at[t, pl.ds(wid * NCHUNK, NCHUNK)],
                        idx_v.at[t])

    def fire(c):
        return [pltpu.async_copy(tables[t].at[idx_v.at[t, c]],
                                 bufs[t][c % 2], sem)
                for t in range(3)]

    iota = lax.iota(jnp.int32, L)
    lane15 = jnp.full((L,), L - 1, jnp.int32)
    inflight = fire(0)
    for c in range(NCHUNK):
        nxt = fire(c + 1) if c + 1 < NCHUNK else []
        for cp in inflight:
            cp.wait()
        g0, g1, g2 = (bufs[t][c % 2] for t in range(3))

        @plsc.parallel_loop(0, GPC, unroll=2)
        def group(k):
            acc = jnp.zeros((L,), jnp.float32)
            for e in range(L):
                b = k * L + e
                p = [g0[b, pl.ds(rc * L, L)] * g1[b, pl.ds(R + rc * L, L)]
                     * g2[b, pl.ds(rc * L, L)] for rc in range(R // L)]
                s = (p[0] + p[1]) + (p[2] + p[3])
                cs = plsc.cumsum(s)
                tot = lax.gather(
                    cs, lane15[:, None],
                    lax.GatherDimensionNumbers(offset_dims=(),
                                               collapsed_slice_dims=(0,),
                                               start_index_map=(0,)),
                    (1,), mode=lax.GatherScatterMode.PROMISE_IN_BOUNDS)
                acc = jnp.where(iota == e, tot, acc)
            out_v[c, pl.ds(k * L, L)] = acc

        inflight = nxt

    pltpu.sync_copy(out_v, out_hbm.at[pl.ds(wid * NCHUNK, NCHUNK)])


@functools.partial(
    pl.kernel,
    out_type=jax.ShapeDtypeStruct((B // RP, RP), jnp.float32),
    mesh=plsc.VectorSubcoreMesh(core_axis_name="c", subcore_axis_name="s",
                                num_cores=NC, num_subcores=NS),
    scratch_types=[
        pltpu.VMEM((3, NCHUNK, CHUNK), jnp.int32),
    ] + [pltpu.VMEM((CHUNK, RP), jnp.float32) for _ in range(6)] + [
        pltpu.VMEM((NCHUNK, CHUNK), jnp.float32),
        pltpu.SemaphoreType.DMA,
    ],
    compiler_params=pltpu.CompilerParams(needs_layout_passes=False,
                                         use_tc_tiling_on_sc=True),
)
def _cp_lookup(idx_hbm, f0_hbm, f1_hbm, f2_hbm, out_hbm,
               idx_v, b0a, b0b, b1a, b1b, b2a, b2b, out_v, sem):
    bufs = ((b0a, b0b), (b1a, b1b), (b2a, b2b))
    _body(idx_hbm, f0_hbm, f1_hbm, f2_hbm, out_hbm,
          idx_v, bufs, out_v, sem)


NROW = 100000
BV = 16384


def _mxu_t(x, eye):
    return lax.dot_general(x, eye, (((0,), (0,)), ((), ())),
                           preferred_element_type=jnp.float32)


def _tr2_body(a_ref, b_ref, eye_ref, out_ref):
    out_ref[:, pl.ds(0, R)] = _mxu_t(a_ref[...], eye_ref[...])
    out_ref[:, pl.ds(R, R)] = _mxu_t(b_ref[...], eye_ref[...])


def _tr1_body(a_ref, eye_ref, out_ref):
    out_ref[:, pl.ds(0, R)] = _mxu_t(a_ref[...], eye_ref[...])


_in_spec = pl.BlockSpec((R, BV), lambda i: (0, i))
_eye_spec = pl.BlockSpec((R, R), lambda i: (0, 0))
_tr_grid = (NROW + BV - 1) // BV
_tr_out = jax.ShapeDtypeStruct((NROW, RP), jnp.float32)
_out_spec = pl.BlockSpec((BV, RP), lambda i: (i, 0))

_transpose2 = pl.pallas_call(
    _tr2_body, grid=(_tr_grid,),
    in_specs=[_in_spec, _in_spec, _eye_spec],
    out_specs=_out_spec, out_shape=_tr_out,
)

_transpose1 = pl.pallas_call(
    _tr1_body, grid=(_tr_grid,),
    in_specs=[_in_spec, _eye_spec],
    out_specs=_out_spec, out_shape=_tr_out,
)


def kernel(input, F0, F1, F2):
    idx = jnp.transpose(input.astype(jnp.int32)).reshape(3, B // CHUNK, CHUNK)
    eye = jnp.eye(R, dtype=jnp.float32)
    g01 = _transpose2(F0.T, F1.T, eye)
    g2 = _transpose1(F2.T, eye)
    out2d = _cp_lookup(idx, g01, g01, g2)
    return out2d.reshape(B)
